# Initial kernel scaffold; baseline (speedup 1.0000x reference)
import jax
import jax.numpy as jnp
from jax.experimental import pallas as pl


def kernel(nfeats, edge_index, efeats, Wa0, ba0, We0, be0, Wa1, ba1, We1, be1):
    raise NotImplementedError("write your pallas kernel here")


# speedup vs baseline: 2.7813x; 2.7813x over previous
import functools

import jax
import jax.numpy as jnp
from jax import lax
from jax.experimental import pallas as pl
from jax.experimental.pallas import tpu as pltpu
from jax.experimental.pallas import tpu_sc as plsc

NC = 2
NS = 16
L = 16

B = 80


def _relu_add_rows(acc, other, n_rows, n_col_vregs):
    def body(r, _):
        for q in range(n_col_vregs):
            a = acc[r, pl.ds(q * L, L)]
            b = other[r, pl.ds(q * L, L)]
            acc[r, pl.ds(q * L, L)] = jnp.maximum(a + b, 0.0)
        return 0
    lax.fori_loop(0, n_rows, body, 0, unroll=2)


def _zero_acc(zn, acc, s, rpt):
    pltpu.sync_copy(zn.at[pl.ds(s * rpt, rpt)], acc.at[pl.ds(s * rpt, rpt)])


def _k1_body(NP, E, DE, nf, srcq, dstq, ef, zn,
             sn0_out, sed_out,
             acc, src_v, dst_v, rows, erows):
    c = lax.axis_index("c")
    s = lax.axis_index("s")
    rpt = NP // NS
    epc = E // (NC * NS)
    nblk = epc // B

    _zero_acc(zn, acc, s, rpt)
    plsc.subcore_barrier()

    def blk_a(j, _):
        base = c * (E // NC) + s * epc + j * B
        pltpu.sync_copy(srcq.at[pl.ds(base, B)], src_v)
        pltpu.sync_copy(dstq.at[pl.ds(base, B)], dst_v)
        pltpu.sync_copy(nf.at[src_v], rows)
        pltpu.sync_copy(rows, acc.at[dst_v], add=True)
        return 0

    lax.fori_loop(0, nblk, blk_a, 0)
    plsc.subcore_barrier()
    pltpu.sync_copy(acc.at[pl.ds(s * rpt, rpt)],
                    sn0_out.at[c, pl.ds(s * rpt, rpt)])
    plsc.subcore_barrier()

    _zero_acc(zn, acc, s, rpt)
    one0 = jnp.where(lax.iota(jnp.int32, L) == 0, 1.0, 0.0).astype(jnp.float32)
    zv = jnp.zeros((L,), jnp.float32)

    def fill(r, _):
        rows[r, pl.ds(DE, L)] = one0
        for q in range(DE // L + 2, 128 // L):
            rows[r, pl.ds(q * L, L)] = zv
        return 0

    lax.fori_loop(0, B, fill, 0, unroll=2)
    plsc.subcore_barrier()

    def blk_b(j, _):
        base = c * (E // NC) + s * epc + j * B
        pltpu.sync_copy(dstq.at[pl.ds(base, B)], dst_v)
        pltpu.sync_copy(ef.at[pl.ds(base, B)], erows)

        def cp(r, _):
            rows[r, pl.ds(0, L)] = erows[r, :]
            return 0

        lax.fori_loop(0, B, cp, 0, unroll=2)
        pltpu.sync_copy(rows, acc.at[dst_v], add=True)
        return 0

    lax.fori_loop(0, nblk, blk_b, 0)
    plsc.subcore_barrier()
    pltpu.sync_copy(acc.at[pl.ds(s * rpt, rpt)],
                    sed_out.at[c, pl.ds(s * rpt, rpt)])


def _k1(NP, E, DE, nf, srcq, dstq, ef, zn):
    mesh = plsc.VectorSubcoreMesh(core_axis_name="c", subcore_axis_name="s",
                                  num_cores=NC, num_subcores=NS)
    kfn = pl.kernel(
        functools.partial(_k1_body, NP, E, DE),
        out_type=(jax.ShapeDtypeStruct((NC, NP, 128), jnp.float32),
                  jax.ShapeDtypeStruct((NC, NP, 128), jnp.float32)),
        mesh=mesh,
        scratch_types=[
            pltpu.VMEM_SHARED((NP, 128), jnp.float32),
            pltpu.VMEM((B,), jnp.int32),
            pltpu.VMEM((B,), jnp.int32),
            pltpu.VMEM((B, 128), jnp.float32),
            pltpu.VMEM((B, 16), jnp.float32),
        ],
        name="egs_k1_layer0_agg",
    )
    return kfn(nf, srcq, dstq, ef, zn)


def _k2_body(NP, E, h1t, a0t, b0t, srcq, dstq, zn,
             sn1_out, se1_out,
             acc, src_v, dst_v, arows, brows):
    c = lax.axis_index("c")
    s = lax.axis_index("s")
    rpt = NP // NS
    epc = E // (NC * NS)
    nblk = epc // B

    _zero_acc(zn, acc, s, rpt)
    plsc.subcore_barrier()

    def blk_a(j, _):
        base = c * (E // NC) + s * epc + j * B
        pltpu.sync_copy(srcq.at[pl.ds(base, B)], src_v)
        pltpu.sync_copy(dstq.at[pl.ds(base, B)], dst_v)
        pltpu.sync_copy(h1t.at[src_v], arows)
        pltpu.sync_copy(arows, acc.at[dst_v], add=True)
        return 0

    lax.fori_loop(0, nblk, blk_a, 0)
    plsc.subcore_barrier()
    pltpu.sync_copy(acc.at[pl.ds(s * rpt, rpt)],
                    sn1_out.at[c, pl.ds(s * rpt, rpt)])
    plsc.subcore_barrier()

    _zero_acc(zn, acc, s, rpt)
    plsc.subcore_barrier()

    def blk_b(j, _):
        base = c * (E // NC) + s * epc + j * B
        pltpu.sync_copy(srcq.at[pl.ds(base, B)], src_v)
        pltpu.sync_copy(dstq.at[pl.ds(base, B)], dst_v)
        pltpu.sync_copy(a0t.at[src_v], arows)
        pltpu.sync_copy(b0t.at[dst_v], brows)
        _relu_add_rows(arows, brows, B, 8)
        pltpu.sync_copy(arows, acc.at[dst_v], add=True)
        return 0

    lax.fori_loop(0, nblk, blk_b, 0)
    plsc.subcore_barrier()
    pltpu.sync_copy(acc.at[pl.ds(s * rpt, rpt)],
                    se1_out.at[c, pl.ds(s * rpt, rpt)])


def _k2(NP, E, h1t, a0t, b0t, srcq, dstq, zn):
    mesh = plsc.VectorSubcoreMesh(core_axis_name="c", subcore_axis_name="s",
                                  num_cores=NC, num_subcores=NS)
    kfn = pl.kernel(
        functools.partial(_k2_body, NP, E),
        out_type=(jax.ShapeDtypeStruct((NC, NP, 128), jnp.float32),
                  jax.ShapeDtypeStruct((NC, NP, 128), jnp.float32)),
        mesh=mesh,
        scratch_types=[
            pltpu.VMEM_SHARED((NP, 128), jnp.float32),
            pltpu.VMEM((B,), jnp.int32),
            pltpu.VMEM((B,), jnp.int32),
            pltpu.VMEM((B, 128), jnp.float32),
            pltpu.VMEM((B, 128), jnp.float32),
        ],
        name="egs_k2_layer1_agg",
    )
    return kfn(h1t, a0t, b0t, srcq, dstq, zn)


def _k3_body(E, H, a1, b1, srcq, dstq, e2_out,
             src_v, dst_v, arows, brows):
    c = lax.axis_index("c")
    s = lax.axis_index("s")
    wid = s * NC + c
    epw = E // (NC * NS)
    nblk = epw // B

    def blk(j, _):
        base = wid * epw + j * B
        pltpu.sync_copy(srcq.at[pl.ds(base, B)], src_v)
        pltpu.sync_copy(dstq.at[pl.ds(base, B)], dst_v)
        pltpu.sync_copy(a1.at[src_v], arows)
        pltpu.sync_copy(b1.at[dst_v], brows)
        _relu_add_rows(arows, brows, B, H // L)
        pltpu.sync_copy(arows, e2_out.at[pl.ds(base, B)])
        return 0

    lax.fori_loop(0, nblk, blk, 0)


def _k3(E, H, a1, b1, srcq, dstq):
    mesh = plsc.VectorSubcoreMesh(core_axis_name="c", subcore_axis_name="s",
                                  num_cores=NC, num_subcores=NS)
    kfn = pl.kernel(
        functools.partial(_k3_body, E, H),
        out_type=jax.ShapeDtypeStruct((E, H), jnp.float32),
        mesh=mesh,
        scratch_types=[
            pltpu.VMEM((B,), jnp.int32),
            pltpu.VMEM((B,), jnp.int32),
            pltpu.VMEM((B, H), jnp.float32),
            pltpu.VMEM((B, H), jnp.float32),
        ],
        name="egs_k3_edge_out",
    )
    return kfn(a1, b1, srcq, dstq)


def _tcA_kernel(DE, nf_ref, sn0_ref, sed_ref, wa_ref, ba_ref, we_ref, be_ref,
                h1_ref, a0_ref, b0_ref):
    D = nf_ref.shape[1]
    sed = sed_ref[0] + sed_ref[1]
    deg = sed[:, DE:DE + 1]
    inv = 1.0 / jnp.maximum(deg, 1.0)
    f32 = jnp.float32
    z = jnp.dot(nf_ref[...], wa_ref[0:D], preferred_element_type=f32)
    sn0 = sn0_ref[0] + sn0_ref[1]
    z += jnp.dot(sn0 * inv, wa_ref[D:2 * D], preferred_element_type=f32)
    z += jnp.dot(sed[:, 0:DE] * inv, wa_ref[2 * D:], preferred_element_type=f32)
    h1 = jnp.maximum(z + ba_ref[...], 0.0)
    h1_ref[...] = h1
    H = we_ref.shape[1]
    a0_ref[...] = jnp.dot(h1, we_ref[0:H], preferred_element_type=f32) + be_ref[...]
    b0_ref[...] = jnp.dot(h1, we_ref[H:], preferred_element_type=f32)


def _tcA(NP, DE, nf_p, sn0, sed, Wa0, ba0, We0, be0):
    H = We0.shape[1]
    RB = NP // 8
    row = pl.BlockSpec((RB, H), lambda i: (i, 0))
    part = pl.BlockSpec((2, RB, H), lambda i: (0, i, 0))
    return pl.pallas_call(
        functools.partial(_tcA_kernel, DE),
        grid=(NP // RB,),
        in_specs=[row, part, part,
                  pl.BlockSpec(Wa0.shape, lambda i: (0, 0)),
                  pl.BlockSpec(ba0.shape, lambda i: (0,)),
                  pl.BlockSpec(We0.shape, lambda i: (0, 0)),
                  pl.BlockSpec(be0.shape, lambda i: (0,))],
        out_specs=(row, row, row),
        out_shape=(jax.ShapeDtypeStruct((NP, H), jnp.float32),
                   jax.ShapeDtypeStruct((NP, H), jnp.float32),
                   jax.ShapeDtypeStruct((NP, H), jnp.float32)),
        name="egs_tcA",
    )(nf_p, sn0, sed, Wa0, ba0, We0, be0)


def _tcB_kernel(DE, h1_ref, sn1_ref, se1_ref, sed_ref, wa_ref, ba_ref, we_ref,
                be_ref, h2_ref, a1_ref, b1_ref):
    deg = (sed_ref[0] + sed_ref[1])[:, DE:DE + 1]
    inv = 1.0 / jnp.maximum(deg, 1.0)
    f32 = jnp.float32
    H = wa_ref.shape[1]
    z = jnp.dot(h1_ref[...], wa_ref[0:H], preferred_element_type=f32)
    sn1 = sn1_ref[0] + sn1_ref[1]
    z += jnp.dot(sn1 * inv, wa_ref[H:2 * H], preferred_element_type=f32)
    se1 = se1_ref[0] + se1_ref[1]
    z += jnp.dot(se1 * inv, wa_ref[2 * H:], preferred_element_type=f32)
    h2 = jnp.maximum(z + ba_ref[...], 0.0)
    h2_ref[...] = h2
    a1_ref[...] = jnp.dot(h2, we_ref[0:H], preferred_element_type=f32) + be_ref[...]
    b1_ref[...] = jnp.dot(h2, we_ref[H:], preferred_element_type=f32)


def _tcB(NP, DE, h1, sn1, se1, sed, Wa1, ba1, We1, be1):
    H = We1.shape[1]
    RB = NP // 8
    row = pl.BlockSpec((RB, H), lambda i: (i, 0))
    part = pl.BlockSpec((2, RB, H), lambda i: (0, i, 0))
    return pl.pallas_call(
        functools.partial(_tcB_kernel, DE),
        grid=(NP // RB,),
        in_specs=[row, part, part, part,
                  pl.BlockSpec(Wa1.shape, lambda i: (0, 0)),
                  pl.BlockSpec(ba1.shape, lambda i: (0,)),
                  pl.BlockSpec(We1.shape, lambda i: (0, 0)),
                  pl.BlockSpec(be1.shape, lambda i: (0,))],
        out_specs=(row, row, row),
        out_shape=(jax.ShapeDtypeStruct((NP, H), jnp.float32),
                   jax.ShapeDtypeStruct((NP, H), jnp.float32),
                   jax.ShapeDtypeStruct((NP, H), jnp.float32)),
        name="egs_tcB",
    )(h1, sn1, se1, sed, Wa1, ba1, We1, be1)


def kernel(nfeats, edge_index, efeats, Wa0, ba0, We0, be0, Wa1, ba1, We1, be1):
    N, D = nfeats.shape
    E = edge_index.shape[1]
    DE = efeats.shape[1]
    H = We0.shape[1]
    assert D == 128 and H == 128 and DE == 16
    NP = ((N + NS * 8 - 1) // (NS * 8)) * (NS * 8)
    assert E % (NC * NS * B) == 0

    srcq = edge_index[0]
    dstq = edge_index[1]
    nf_p = jnp.pad(nfeats, ((0, NP - N), (0, 0)))
    zn = jnp.zeros((NP, 128), jnp.float32)

    sn0, sed = _k1(NP, E, DE, nf_p, srcq, dstq, efeats, zn)
    h1, a0, b0 = _tcA(NP, DE, nf_p, sn0, sed, Wa0, ba0, We0, be0)
    sn1, se1 = _k2(NP, E, h1, a0, b0, srcq, dstq, zn)
    h2, a1, b1 = _tcB(NP, DE, h1, sn1, se1, sed, Wa1, ba1, We1, be1)
    e2 = _k3(E, H, a1, b1, srcq, dstq)
    return (h2[:N], e2)


# speedup vs baseline: 5.1580x; 1.8545x over previous
import functools

import jax
import jax.numpy as jnp
from jax import lax
from jax.experimental import pallas as pl
from jax.experimental.pallas import tpu as pltpu
from jax.experimental.pallas import tpu_sc as plsc

NC = 2
NS = 16
L = 16

B = 80


def _relu_add_rows(acc, other, slot, n_rows, n_col_vregs):
    def body(r, _):
        for q in range(n_col_vregs):
            a = acc[slot, r, pl.ds(q * L, L)]
            b = other[slot, r, pl.ds(q * L, L)]
            acc[slot, r, pl.ds(q * L, L)] = jnp.maximum(a + b, 0.0)
        return 0
    lax.fori_loop(0, n_rows, body, 0, unroll=2)


def _zero_acc(zn, acc, s, rpt):
    pltpu.sync_copy(zn.at[pl.ds(s * rpt, rpt)], acc.at[pl.ds(s * rpt, rpt)])


def _wait(src_dummy, dst_dummy, sem):
    pltpu.make_async_copy(src_dummy, dst_dummy, sem).wait()


def _pipe(nblk, fetch, compute, scatter, wait_fetch, wait_scatter,
          fetch_idx=None, wait_idx=None):
    def step(j, cur, nxt):
        if fetch_idx is not None:
            @pl.when(j + 2 < nblk)
            def _():
                fetch_idx(j + 2, cur)

        @pl.when(j >= 1)
        def _():
            wait_scatter(j - 1, nxt)

        @pl.when(j + 1 < nblk)
        def _():
            if wait_idx is not None:
                wait_idx(nxt)
            fetch(j + 1, nxt)
        wait_fetch(j, cur)
        if compute is not None:
            compute(j, cur)
        scatter(j, cur)

    if fetch_idx is not None:
        fetch_idx(0, 0)
        fetch_idx(1, 1)
        wait_idx(0)
    fetch(0, 0)

    def pair(g, _):
        j = 2 * g
        step(j, 0, 1)
        step(j + 1, 1, 0)
        return 0

    lax.fori_loop(0, nblk // 2, pair, 0)
    if nblk % 2 == 1:
        step(jnp.int32(nblk - 1), 0, 1)
    wait_scatter(jnp.int32(nblk - 1), (nblk - 1) % 2)


def _k1_body(NP, E, DE, nf, srcq, dstq, ef, zn,
             sn0_out, sed_out,
             acc, isrc, idst, rows, erows,
             isb0, isb1, idb0, idb1,
             sem_i0, sem_i1, sem_g0, sem_g1, sem_s0, sem_s1):
    c = lax.axis_index("c")
    s = lax.axis_index("s")
    rpt = NP // NS
    epc = E // (NC * NS)
    nblk = epc // B
    ebase = c * (E // NC) + s * epc
    sem_i = (sem_i0, sem_i1)
    sem_g = (sem_g0, sem_g1)
    sem_s = (sem_s0, sem_s1)
    isb = (isb0, isb1)
    idb = (idb0, idb1)

    def fetch_idx(j, p):
        pltpu.async_copy(srcq.at[pl.ds(ebase + j * B, B)], isrc.at[j & 3], sem_i[p])
        pltpu.async_copy(dstq.at[pl.ds(ebase + j * B, B)], idst.at[j & 3], sem_i[p])

    def wait_idx(p):
        _wait(srcq.at[pl.ds(0, B)], isrc.at[0], sem_i[p])
        _wait(srcq.at[pl.ds(0, B)], idst.at[0], sem_i[p])

    def copy_row(src2d, j, dstbuf):
        def body(i, _):
            dstbuf[pl.ds(i * L, L)] = src2d[j & 3, pl.ds(i * L, L)]
            return 0
        lax.fori_loop(0, B // L, body, 0, unroll=B // L)

    def scatter(j, slot):
        pltpu.async_copy(rows.at[slot], acc.at[idb[slot]], sem_s[slot],
                         add=True)

    def wait_scatter(j, slot):
        _wait(rows.at[slot], acc.at[idb[slot]], sem_s[slot])

    _zero_acc(zn, acc, s, rpt)
    plsc.subcore_barrier()

    def fetch_a(j, slot):
        copy_row(isrc, j, isb[slot])
        copy_row(idst, j, idb[slot])
        pltpu.async_copy(nf.at[isb[slot]], rows.at[slot], sem_g[slot])

    def wait_fetch_a(j, slot):
        _wait(nf.at[isb[slot]], rows.at[slot], sem_g[slot])

    _pipe(nblk, fetch_a, None, scatter, wait_fetch_a, wait_scatter,
          fetch_idx, wait_idx)

    plsc.subcore_barrier()
    pltpu.sync_copy(acc.at[pl.ds(s * rpt, rpt)],
                    sn0_out.at[c, pl.ds(s * rpt, rpt)])
    plsc.subcore_barrier()

    _zero_acc(zn, acc, s, rpt)
    one0 = jnp.where(lax.iota(jnp.int32, L) == 0, 1.0, 0.0).astype(jnp.float32)
    zv = jnp.zeros((L,), jnp.float32)

    def fill(r, _):
        for slot in (0, 1):
            rows[slot, r, pl.ds(DE, L)] = one0
            for q in range(DE // L + 1, 128 // L):
                rows[slot, r, pl.ds(q * L, L)] = zv
        return 0

    lax.fori_loop(0, B, fill, 0, unroll=2)
    plsc.subcore_barrier()

    def fetch_b(j, slot):
        copy_row(idst, j, idb[slot])
        pltpu.async_copy(ef.at[pl.ds(ebase + j * B, B)], erows.at[slot],
                         sem_g[slot])

    def wait_fetch_b(j, slot):
        _wait(ef.at[pl.ds(0, B)], erows.at[slot], sem_g[slot])

    def compute_b(j, slot):
        def cp(r, _):
            rows[slot, r, pl.ds(0, L)] = erows[slot, r, :]
            return 0
        lax.fori_loop(0, B, cp, 0, unroll=4)

    _pipe(nblk, fetch_b, compute_b, scatter, wait_fetch_b, wait_scatter,
          fetch_idx, wait_idx)

    plsc.subcore_barrier()
    pltpu.sync_copy(acc.at[pl.ds(s * rpt, rpt)],
                    sed_out.at[c, pl.ds(s * rpt, rpt)])


def _k1(NP, E, DE, nf, srcq, dstq, ef, zn):
    mesh = plsc.VectorSubcoreMesh(core_axis_name="c", subcore_axis_name="s",
                                  num_cores=NC, num_subcores=NS)
    nblk = E // (NC * NS * B)
    kfn = pl.kernel(
        functools.partial(_k1_body, NP, E, DE),
        out_type=(jax.ShapeDtypeStruct((NC, NP, 128), jnp.float32),
                  jax.ShapeDtypeStruct((NC, NP, 128), jnp.float32)),
        mesh=mesh,
        scratch_types=[
            pltpu.VMEM_SHARED((NP, 128), jnp.float32),
            pltpu.VMEM((4, B), jnp.int32),
            pltpu.VMEM((4, B), jnp.int32),
            pltpu.VMEM((2, B, 128), jnp.float32),
            pltpu.VMEM((2, B, 16), jnp.float32),
            pltpu.VMEM((B,), jnp.int32),
            pltpu.VMEM((B,), jnp.int32),
            pltpu.VMEM((B,), jnp.int32),
            pltpu.VMEM((B,), jnp.int32),
            pltpu.SemaphoreType.DMA,
            pltpu.SemaphoreType.DMA,
            pltpu.SemaphoreType.DMA,
            pltpu.SemaphoreType.DMA,
            pltpu.SemaphoreType.DMA,
            pltpu.SemaphoreType.DMA,
        ],
        name="egs_k1_layer0_agg",
    )
    return kfn(nf, srcq, dstq, ef, zn)


def _k2_body(NP, E, h1t, a0t, b0t, srcq, dstq, zn,
             sn1_out, se1_out,
             acc, isrc, idst, arows, brows,
             isb0, isb1, idb0, idb1,
             sem_i0, sem_i1, sem_g0, sem_g1, sem_s0, sem_s1):
    c = lax.axis_index("c")
    s = lax.axis_index("s")
    rpt = NP // NS
    epc = E // (NC * NS)
    nblk = epc // B
    ebase = c * (E // NC) + s * epc
    sem_i = (sem_i0, sem_i1)
    sem_g = (sem_g0, sem_g1)
    sem_s = (sem_s0, sem_s1)
    isb = (isb0, isb1)
    idb = (idb0, idb1)

    def fetch_idx(j, p):
        pltpu.async_copy(srcq.at[pl.ds(ebase + j * B, B)], isrc.at[j & 3], sem_i[p])
        pltpu.async_copy(dstq.at[pl.ds(ebase + j * B, B)], idst.at[j & 3], sem_i[p])

    def wait_idx(p):
        _wait(srcq.at[pl.ds(0, B)], isrc.at[0], sem_i[p])
        _wait(srcq.at[pl.ds(0, B)], idst.at[0], sem_i[p])

    def copy_row(src2d, j, dstbuf):
        def body(i, _):
            dstbuf[pl.ds(i * L, L)] = src2d[j & 3, pl.ds(i * L, L)]
            return 0
        lax.fori_loop(0, B // L, body, 0, unroll=B // L)

    def scatter(j, slot):
        pltpu.async_copy(arows.at[slot], acc.at[idb[slot]], sem_s[slot],
                         add=True)

    def wait_scatter(j, slot):
        _wait(arows.at[slot], acc.at[idb[slot]], sem_s[slot])

    _zero_acc(zn, acc, s, rpt)
    plsc.subcore_barrier()

    def fetch_a(j, slot):
        copy_row(isrc, j, isb[slot])
        copy_row(idst, j, idb[slot])
        pltpu.async_copy(h1t.at[isb[slot]], arows.at[slot], sem_g[slot])

    def wait_fetch_a(j, slot):
        _wait(h1t.at[isb[slot]], arows.at[slot], sem_g[slot])

    _pipe(nblk, fetch_a, None, scatter, wait_fetch_a, wait_scatter,
          fetch_idx, wait_idx)

    plsc.subcore_barrier()
    pltpu.sync_copy(acc.at[pl.ds(s * rpt, rpt)],
                    sn1_out.at[c, pl.ds(s * rpt, rpt)])
    plsc.subcore_barrier()

    _zero_acc(zn, acc, s, rpt)
    plsc.subcore_barrier()

    def fetch_b(j, slot):
        copy_row(isrc, j, isb[slot])
        copy_row(idst, j, idb[slot])
        pltpu.async_copy(a0t.at[isb[slot]], arows.at[slot], sem_g[slot])
        pltpu.async_copy(b0t.at[idb[slot]], brows.at[slot], sem_g[slot])

    def wait_fetch_b(j, slot):
        _wait(a0t.at[isb[slot]], arows.at[slot], sem_g[slot])
        _wait(b0t.at[idb[slot]], brows.at[slot], sem_g[slot])

    def compute_b(j, slot):
        _relu_add_rows(arows, brows, slot, B, 8)

    _pipe(nblk, fetch_b, compute_b, scatter, wait_fetch_b, wait_scatter,
          fetch_idx, wait_idx)

    plsc.subcore_barrier()
    pltpu.sync_copy(acc.at[pl.ds(s * rpt, rpt)],
                    se1_out.at[c, pl.ds(s * rpt, rpt)])


def _k2(NP, E, h1t, a0t, b0t, srcq, dstq, zn):
    mesh = plsc.VectorSubcoreMesh(core_axis_name="c", subcore_axis_name="s",
                                  num_cores=NC, num_subcores=NS)
    nblk = E // (NC * NS * B)
    kfn = pl.kernel(
        functools.partial(_k2_body, NP, E),
        out_type=(jax.ShapeDtypeStruct((NC, NP, 128), jnp.float32),
                  jax.ShapeDtypeStruct((NC, NP, 128), jnp.float32)),
        mesh=mesh,
        scratch_types=[
            pltpu.VMEM_SHARED((NP, 128), jnp.float32),
            pltpu.VMEM((4, B), jnp.int32),
            pltpu.VMEM((4, B), jnp.int32),
            pltpu.VMEM((2, B, 128), jnp.float32),
            pltpu.VMEM((2, B, 128), jnp.float32),
            pltpu.VMEM((B,), jnp.int32),
            pltpu.VMEM((B,), jnp.int32),
            pltpu.VMEM((B,), jnp.int32),
            pltpu.VMEM((B,), jnp.int32),
            pltpu.SemaphoreType.DMA,
            pltpu.SemaphoreType.DMA,
            pltpu.SemaphoreType.DMA,
            pltpu.SemaphoreType.DMA,
            pltpu.SemaphoreType.DMA,
            pltpu.SemaphoreType.DMA,
        ],
        name="egs_k2_layer1_agg",
    )
    return kfn(h1t, a0t, b0t, srcq, dstq, zn)


def _k3_body(E, H, a1, b1, srcq, dstq, e2_out,
             isrc, idst, arows, brows,
             isb0, isb1, idb0, idb1,
             sem_i0, sem_i1, sem_g0, sem_g1, sem_s0, sem_s1):
    c = lax.axis_index("c")
    s = lax.axis_index("s")
    wid = s * NC + c
    epw = E // (NC * NS)
    nblk = epw // B
    ebase = wid * epw
    sem_i = (sem_i0, sem_i1)
    sem_g = (sem_g0, sem_g1)
    sem_s = (sem_s0, sem_s1)
    isb = (isb0, isb1)
    idb = (idb0, idb1)

    def fetch_idx(j, p):
        pltpu.async_copy(srcq.at[pl.ds(ebase + j * B, B)], isrc.at[j & 3], sem_i[p])
        pltpu.async_copy(dstq.at[pl.ds(ebase + j * B, B)], idst.at[j & 3], sem_i[p])

    def wait_idx(p):
        _wait(srcq.at[pl.ds(0, B)], isrc.at[0], sem_i[p])
        _wait(srcq.at[pl.ds(0, B)], idst.at[0], sem_i[p])

    def copy_row(src2d, j, dstbuf):
        def body(i, _):
            dstbuf[pl.ds(i * L, L)] = src2d[j & 3, pl.ds(i * L, L)]
            return 0
        lax.fori_loop(0, B // L, body, 0, unroll=B // L)

    def fetch(j, slot):
        copy_row(isrc, j, isb[slot])
        copy_row(idst, j, idb[slot])
        pltpu.async_copy(a1.at[isb[slot]], arows.at[slot], sem_g[slot])
        pltpu.async_copy(b1.at[idb[slot]], brows.at[slot], sem_g[slot])

    def wait_fetch(j, slot):
        _wait(a1.at[isb[slot]], arows.at[slot], sem_g[slot])
        _wait(b1.at[idb[slot]], brows.at[slot], sem_g[slot])

    def compute(j, slot):
        _relu_add_rows(arows, brows, slot, B, H // L)

    def store(j, slot):
        pltpu.async_copy(arows.at[slot], e2_out.at[pl.ds(ebase + j * B, B)],
                         sem_s[slot])

    def wait_store(j, slot):
        _wait(arows.at[slot], e2_out.at[pl.ds(ebase + j * B, B)], sem_s[slot])

    _pipe(nblk, fetch, compute, store, wait_fetch, wait_store,
          fetch_idx, wait_idx)


def _k3(E, H, a1, b1, srcq, dstq):
    mesh = plsc.VectorSubcoreMesh(core_axis_name="c", subcore_axis_name="s",
                                  num_cores=NC, num_subcores=NS)
    nblk = E // (NC * NS * B)
    kfn = pl.kernel(
        functools.partial(_k3_body, E, H),
        out_type=jax.ShapeDtypeStruct((E, H), jnp.float32),
        mesh=mesh,
        scratch_types=[
            pltpu.VMEM((4, B), jnp.int32),
            pltpu.VMEM((4, B), jnp.int32),
            pltpu.VMEM((2, B, H), jnp.float32),
            pltpu.VMEM((2, B, H), jnp.float32),
            pltpu.VMEM((B,), jnp.int32),
            pltpu.VMEM((B,), jnp.int32),
            pltpu.VMEM((B,), jnp.int32),
            pltpu.VMEM((B,), jnp.int32),
            pltpu.SemaphoreType.DMA,
            pltpu.SemaphoreType.DMA,
            pltpu.SemaphoreType.DMA,
            pltpu.SemaphoreType.DMA,
            pltpu.SemaphoreType.DMA,
            pltpu.SemaphoreType.DMA,
        ],
        name="egs_k3_edge_out",
    )
    return kfn(a1, b1, srcq, dstq)


def _tcA_kernel(DE, nf_ref, sn0_ref, sed_ref, wa_ref, ba_ref, we_ref, be_ref,
                h1_ref, a0_ref, b0_ref):
    D = nf_ref.shape[1]
    sed = sed_ref[0] + sed_ref[1]
    deg = sed[:, DE:DE + 1]
    inv = 1.0 / jnp.maximum(deg, 1.0)
    f32 = jnp.float32
    z = jnp.dot(nf_ref[...], wa_ref[0:D], preferred_element_type=f32)
    sn0 = sn0_ref[0] + sn0_ref[1]
    z += jnp.dot(sn0 * inv, wa_ref[D:2 * D], preferred_element_type=f32)
    z += jnp.dot(sed[:, 0:DE] * inv, wa_ref[2 * D:], preferred_element_type=f32)
    h1 = jnp.maximum(z + ba_ref[...], 0.0)
    h1_ref[...] = h1
    H = we_ref.shape[1]
    a0_ref[...] = jnp.dot(h1, we_ref[0:H], preferred_element_type=f32) + be_ref[...]
    b0_ref[...] = jnp.dot(h1, we_ref[H:], preferred_element_type=f32)


def _tcA(NP, DE, nf_p, sn0, sed, Wa0, ba0, We0, be0):
    H = We0.shape[1]
    RB = NP // 8
    row = pl.BlockSpec((RB, H), lambda i: (i, 0))
    part = pl.BlockSpec((2, RB, H), lambda i: (0, i, 0))
    return pl.pallas_call(
        functools.partial(_tcA_kernel, DE),
        grid=(NP // RB,),
        in_specs=[row, part, part,
                  pl.BlockSpec(Wa0.shape, lambda i: (0, 0)),
                  pl.BlockSpec(ba0.shape, lambda i: (0,)),
                  pl.BlockSpec(We0.shape, lambda i: (0, 0)),
                  pl.BlockSpec(be0.shape, lambda i: (0,))],
        out_specs=(row, row, row),
        out_shape=(jax.ShapeDtypeStruct((NP, H), jnp.float32),
                   jax.ShapeDtypeStruct((NP, H), jnp.float32),
                   jax.ShapeDtypeStruct((NP, H), jnp.float32)),
        name="egs_tcA",
    )(nf_p, sn0, sed, Wa0, ba0, We0, be0)


def _tcB_kernel(DE, h1_ref, sn1_ref, se1_ref, sed_ref, wa_ref, ba_ref, we_ref,
                be_ref, h2_ref, a1_ref, b1_ref):
    deg = (sed_ref[0] + sed_ref[1])[:, DE:DE + 1]
    inv = 1.0 / jnp.maximum(deg, 1.0)
    f32 = jnp.float32
    H = wa_ref.shape[1]
    z = jnp.dot(h1_ref[...], wa_ref[0:H], preferred_element_type=f32)
    sn1 = sn1_ref[0] + sn1_ref[1]
    z += jnp.dot(sn1 * inv, wa_ref[H:2 * H], preferred_element_type=f32)
    se1 = se1_ref[0] + se1_ref[1]
    z += jnp.dot(se1 * inv, wa_ref[2 * H:], preferred_element_type=f32)
    h2 = jnp.maximum(z + ba_ref[...], 0.0)
    h2_ref[...] = h2
    a1_ref[...] = jnp.dot(h2, we_ref[0:H], preferred_element_type=f32) + be_ref[...]
    b1_ref[...] = jnp.dot(h2, we_ref[H:], preferred_element_type=f32)


def _tcB(NP, DE, h1, sn1, se1, sed, Wa1, ba1, We1, be1):
    H = We1.shape[1]
    RB = NP // 8
    row = pl.BlockSpec((RB, H), lambda i: (i, 0))
    part = pl.BlockSpec((2, RB, H), lambda i: (0, i, 0))
    return pl.pallas_call(
        functools.partial(_tcB_kernel, DE),
        grid=(NP // RB,),
        in_specs=[row, part, part, part,
                  pl.BlockSpec(Wa1.shape, lambda i: (0, 0)),
                  pl.BlockSpec(ba1.shape, lambda i: (0,)),
                  pl.BlockSpec(We1.shape, lambda i: (0, 0)),
                  pl.BlockSpec(be1.shape, lambda i: (0,))],
        out_specs=(row, row, row),
        out_shape=(jax.ShapeDtypeStruct((NP, H), jnp.float32),
                   jax.ShapeDtypeStruct((NP, H), jnp.float32),
                   jax.ShapeDtypeStruct((NP, H), jnp.float32)),
        name="egs_tcB",
    )(h1, sn1, se1, sed, Wa1, ba1, We1, be1)


def kernel(nfeats, edge_index, efeats, Wa0, ba0, We0, be0, Wa1, ba1, We1, be1):
    N, D = nfeats.shape
    E = edge_index.shape[1]
    DE = efeats.shape[1]
    H = We0.shape[1]
    assert D == 128 and H == 128 and DE == 16
    NP = ((N + NS * 8 - 1) // (NS * 8)) * (NS * 8)
    assert E % (NC * NS * B) == 0

    srcq = edge_index[0]
    dstq = edge_index[1]
    nf_p = jnp.pad(nfeats, ((0, NP - N), (0, 0)))
    zn = jnp.zeros((NP, 128), jnp.float32)

    sn0, sed = _k1(NP, E, DE, nf_p, srcq, dstq, efeats, zn)
    h1, a0, b0 = _tcA(NP, DE, nf_p, sn0, sed, Wa0, ba0, We0, be0)
    sn1, se1 = _k2(NP, E, h1, a0, b0, srcq, dstq, zn)
    h2, a1, b1 = _tcB(NP, DE, h1, sn1, se1, sed, Wa1, ba1, We1, be1)
    e2 = _k3(E, H, a1, b1, srcq, dstq)
    return (h2[:N], e2)


# speedup vs baseline: 5.7641x; 1.1175x over previous
import functools

import jax
import jax.numpy as jnp
from jax import lax
from jax.experimental import pallas as pl
from jax.experimental.pallas import tpu as pltpu
from jax.experimental.pallas import tpu_sc as plsc

NC = 2
NS = 16
L = 16

B = 80


def _relu_add_rows(acc, other, slot, n_rows, n_col_vregs):
    def body(r, _):
        for q in range(n_col_vregs):
            a = acc[slot, r, pl.ds(q * L, L)]
            b = other[slot, r, pl.ds(q * L, L)]
            acc[slot, r, pl.ds(q * L, L)] = jnp.maximum(a + b, 0.0)
        return 0
    lax.fori_loop(0, n_rows, body, 0, unroll=2)


def _zero_acc(zn, acc, s, rpt):
    pltpu.sync_copy(zn.at[pl.ds(s * rpt, rpt)], acc.at[pl.ds(s * rpt, rpt)])


def _wait(src_dummy, dst_dummy, sem):
    pltpu.make_async_copy(src_dummy, dst_dummy, sem).wait()


def _pipe(nblk, fetch, compute, scatter, wait_fetch, wait_scatter,
          fetch_idx=None, wait_idx=None):
    def step(j, cur, nxt):
        if fetch_idx is not None:
            @pl.when(j + 2 < nblk)
            def _():
                fetch_idx(j + 2, cur)

        @pl.when(j >= 1)
        def _():
            wait_scatter(j - 1, nxt)

        @pl.when(j + 1 < nblk)
        def _():
            if wait_idx is not None:
                wait_idx(nxt)
            fetch(j + 1, nxt)
        wait_fetch(j, cur)
        if compute is not None:
            compute(j, cur)
        scatter(j, cur)

    if fetch_idx is not None:
        fetch_idx(0, 0)
        fetch_idx(1, 1)
        wait_idx(0)
    fetch(0, 0)

    def pair(g, _):
        j = 2 * g
        step(j, 0, 1)
        step(j + 1, 1, 0)
        return 0

    lax.fori_loop(0, nblk // 2, pair, 0)
    if nblk % 2 == 1:
        step(jnp.int32(nblk - 1), 0, 1)
    wait_scatter(jnp.int32(nblk - 1), (nblk - 1) % 2)


def _k1_body(NP, E, DE, nf, srcq, dstq, ef, zn,
             sn0_out, sed_out,
             acc, isrc, idst, rows, erows,
             isb0, isb1, idb0, idb1,
             sem_i0, sem_i1, sem_g0, sem_g1, sem_s0, sem_s1):
    c = lax.axis_index("c")
    s = lax.axis_index("s")
    rpt = NP // NS
    epc = E // (NC * NS)
    nblk = epc // B
    ebase = c * (E // NC) + s * epc
    sem_i = (sem_i0, sem_i1)
    sem_g = (sem_g0, sem_g1)
    sem_s = (sem_s0, sem_s1)
    isb = (isb0, isb1)
    idb = (idb0, idb1)

    def fetch_idx(j, p):
        pltpu.async_copy(srcq.at[pl.ds(ebase + j * B, B)], isrc.at[j & 3], sem_i[p])
        pltpu.async_copy(dstq.at[pl.ds(ebase + j * B, B)], idst.at[j & 3], sem_i[p])

    def wait_idx(p):
        _wait(srcq.at[pl.ds(0, B)], isrc.at[0], sem_i[p])
        _wait(srcq.at[pl.ds(0, B)], idst.at[0], sem_i[p])

    def copy_row(src2d, j, dstbuf):
        def body(i, _):
            dstbuf[pl.ds(i * L, L)] = src2d[j & 3, pl.ds(i * L, L)]
            return 0
        lax.fori_loop(0, B // L, body, 0, unroll=B // L)

    def scatter(j, slot):
        pltpu.async_copy(rows.at[slot], acc.at[idb[slot]], sem_s[slot],
                         add=True)

    def wait_scatter(j, slot):
        _wait(rows.at[slot], acc.at[idb[slot]], sem_s[slot])

    _zero_acc(zn, acc, s, rpt)
    plsc.subcore_barrier()

    def fetch_a(j, slot):
        copy_row(isrc, j, isb[slot])
        copy_row(idst, j, idb[slot])
        pltpu.async_copy(nf.at[isb[slot]], rows.at[slot], sem_g[slot])

    def wait_fetch_a(j, slot):
        _wait(nf.at[isb[slot]], rows.at[slot], sem_g[slot])

    _pipe(nblk, fetch_a, None, scatter, wait_fetch_a, wait_scatter,
          fetch_idx, wait_idx)

    plsc.subcore_barrier()
    pltpu.sync_copy(acc.at[pl.ds(s * rpt, rpt)],
                    sn0_out.at[c, pl.ds(s * rpt, rpt)])
    plsc.subcore_barrier()

    _zero_acc(zn, acc, s, rpt)
    one0 = jnp.where(lax.iota(jnp.int32, L) == 0, 1.0, 0.0).astype(jnp.float32)
    zv = jnp.zeros((L,), jnp.float32)

    def fill(r, _):
        for slot in (0, 1):
            rows[slot, r, pl.ds(DE, L)] = one0
            for q in range(DE // L + 1, 128 // L):
                rows[slot, r, pl.ds(q * L, L)] = zv
        return 0

    lax.fori_loop(0, B, fill, 0, unroll=2)
    plsc.subcore_barrier()

    def fetch_b(j, slot):
        copy_row(idst, j, idb[slot])
        pltpu.async_copy(ef.at[pl.ds(ebase + j * B, B)], erows.at[slot],
                         sem_g[slot])

    def wait_fetch_b(j, slot):
        _wait(ef.at[pl.ds(0, B)], erows.at[slot], sem_g[slot])

    def compute_b(j, slot):
        def cp(r, _):
            rows[slot, r, pl.ds(0, L)] = erows[slot, r, :]
            return 0
        lax.fori_loop(0, B, cp, 0, unroll=4)

    _pipe(nblk, fetch_b, compute_b, scatter, wait_fetch_b, wait_scatter,
          fetch_idx, wait_idx)

    plsc.subcore_barrier()
    pltpu.sync_copy(acc.at[pl.ds(s * rpt, rpt)],
                    sed_out.at[c, pl.ds(s * rpt, rpt)])


def _k1(NP, E, DE, nf, srcq, dstq, ef, zn):
    mesh = plsc.VectorSubcoreMesh(core_axis_name="c", subcore_axis_name="s",
                                  num_cores=NC, num_subcores=NS)
    nblk = E // (NC * NS * B)
    kfn = pl.kernel(
        functools.partial(_k1_body, NP, E, DE),
        out_type=(jax.ShapeDtypeStruct((NC, NP, 128), jnp.float32),
                  jax.ShapeDtypeStruct((NC, NP, 128), jnp.float32)),
        mesh=mesh,
        scratch_types=[
            pltpu.VMEM_SHARED((NP, 128), jnp.float32),
            pltpu.VMEM((4, B), jnp.int32),
            pltpu.VMEM((4, B), jnp.int32),
            pltpu.VMEM((2, B, 128), jnp.float32),
            pltpu.VMEM((2, B, 16), jnp.float32),
            pltpu.VMEM((B,), jnp.int32),
            pltpu.VMEM((B,), jnp.int32),
            pltpu.VMEM((B,), jnp.int32),
            pltpu.VMEM((B,), jnp.int32),
            pltpu.SemaphoreType.DMA,
            pltpu.SemaphoreType.DMA,
            pltpu.SemaphoreType.DMA,
            pltpu.SemaphoreType.DMA,
            pltpu.SemaphoreType.DMA,
            pltpu.SemaphoreType.DMA,
        ],
        name="egs_k1_layer0_agg",
    )
    return kfn(nf, srcq, dstq, ef, zn)


def _k2_body(NP, E, h1t, a0t, b0t, srcq, dstq, zn,
             sn1_out, se1_out,
             acc, isrc, idst, arows, brows,
             isb0, isb1, idb0, idb1,
             sem_i0, sem_i1, sem_g0, sem_g1, sem_s0, sem_s1):
    c = lax.axis_index("c")
    s = lax.axis_index("s")
    rpt = NP // NS
    epc = E // (NC * NS)
    nblk = epc // B
    ebase = c * (E // NC) + s * epc
    sem_i = (sem_i0, sem_i1)
    sem_g = (sem_g0, sem_g1)
    sem_s = (sem_s0, sem_s1)
    isb = (isb0, isb1)
    idb = (idb0, idb1)

    def fetch_idx(j, p):
        pltpu.async_copy(srcq.at[pl.ds(ebase + j * B, B)], isrc.at[j & 3], sem_i[p])
        pltpu.async_copy(dstq.at[pl.ds(ebase + j * B, B)], idst.at[j & 3], sem_i[p])

    def wait_idx(p):
        _wait(srcq.at[pl.ds(0, B)], isrc.at[0], sem_i[p])
        _wait(srcq.at[pl.ds(0, B)], idst.at[0], sem_i[p])

    def copy_row(src2d, j, dstbuf):
        def body(i, _):
            dstbuf[pl.ds(i * L, L)] = src2d[j & 3, pl.ds(i * L, L)]
            return 0
        lax.fori_loop(0, B // L, body, 0, unroll=B // L)

    def scatter(j, slot):
        pltpu.async_copy(arows.at[slot], acc.at[idb[slot]], sem_s[slot],
                         add=True)

    def wait_scatter(j, slot):
        _wait(arows.at[slot], acc.at[idb[slot]], sem_s[slot])

    _zero_acc(zn, acc, s, rpt)
    plsc.subcore_barrier()

    def fetch_a(j, slot):
        copy_row(isrc, j, isb[slot])
        copy_row(idst, j, idb[slot])
        pltpu.async_copy(h1t.at[isb[slot]], arows.at[slot], sem_g[slot])

    def wait_fetch_a(j, slot):
        _wait(h1t.at[isb[slot]], arows.at[slot], sem_g[slot])

    _pipe(nblk, fetch_a, None, scatter, wait_fetch_a, wait_scatter,
          fetch_idx, wait_idx)

    plsc.subcore_barrier()
    pltpu.sync_copy(acc.at[pl.ds(s * rpt, rpt)],
                    sn1_out.at[c, pl.ds(s * rpt, rpt)])
    plsc.subcore_barrier()

    _zero_acc(zn, acc, s, rpt)
    plsc.subcore_barrier()

    def fetch_b(j, slot):
        copy_row(isrc, j, isb[slot])
        copy_row(idst, j, idb[slot])
        pltpu.async_copy(a0t.at[isb[slot]], arows.at[slot], sem_g[slot])
        pltpu.async_copy(b0t.at[idb[slot]], brows.at[slot], sem_g[slot])

    def wait_fetch_b(j, slot):
        _wait(a0t.at[isb[slot]], arows.at[slot], sem_g[slot])
        _wait(b0t.at[idb[slot]], brows.at[slot], sem_g[slot])

    def compute_b(j, slot):
        _relu_add_rows(arows, brows, slot, B, 8)

    _pipe(nblk, fetch_b, compute_b, scatter, wait_fetch_b, wait_scatter,
          fetch_idx, wait_idx)

    plsc.subcore_barrier()
    pltpu.sync_copy(acc.at[pl.ds(s * rpt, rpt)],
                    se1_out.at[c, pl.ds(s * rpt, rpt)])


def _k2(NP, E, h1t, a0t, b0t, srcq, dstq, zn):
    mesh = plsc.VectorSubcoreMesh(core_axis_name="c", subcore_axis_name="s",
                                  num_cores=NC, num_subcores=NS)
    nblk = E // (NC * NS * B)
    kfn = pl.kernel(
        functools.partial(_k2_body, NP, E),
        out_type=(jax.ShapeDtypeStruct((NC, NP, 128), jnp.float32),
                  jax.ShapeDtypeStruct((NC, NP, 128), jnp.float32)),
        mesh=mesh,
        scratch_types=[
            pltpu.VMEM_SHARED((NP, 128), jnp.float32),
            pltpu.VMEM((4, B), jnp.int32),
            pltpu.VMEM((4, B), jnp.int32),
            pltpu.VMEM((2, B, 128), jnp.float32),
            pltpu.VMEM((2, B, 128), jnp.float32),
            pltpu.VMEM((B,), jnp.int32),
            pltpu.VMEM((B,), jnp.int32),
            pltpu.VMEM((B,), jnp.int32),
            pltpu.VMEM((B,), jnp.int32),
            pltpu.SemaphoreType.DMA,
            pltpu.SemaphoreType.DMA,
            pltpu.SemaphoreType.DMA,
            pltpu.SemaphoreType.DMA,
            pltpu.SemaphoreType.DMA,
            pltpu.SemaphoreType.DMA,
        ],
        name="egs_k2_layer1_agg",
    )
    return kfn(h1t, a0t, b0t, srcq, dstq, zn)


_D3 = 4


def _k3_body(E, H, a1, b1, srcq, dstq, e2_out,
             isrc, idst, arows, brows, *rest):
    c = lax.axis_index("c")
    s = lax.axis_index("s")
    wid = s * NC + c
    epw = E // (NC * NS)
    nblk = epw // B
    ebase = wid * epw
    D = _D3
    isb = rest[0:D]
    idb = rest[D:2 * D]
    sems = rest[2 * D:]
    sem_i = sems[0:4]
    sem_g = sems[4:4 + D]
    sem_s = sems[4 + D:4 + 2 * D]

    def fetch_idx(j, p):
        pltpu.async_copy(srcq.at[pl.ds(ebase + j * B, B)], isrc.at[j & 7], sem_i[p])
        pltpu.async_copy(dstq.at[pl.ds(ebase + j * B, B)], idst.at[j & 7], sem_i[p])

    def wait_idx(p):
        _wait(srcq.at[pl.ds(0, B)], isrc.at[0], sem_i[p])
        _wait(srcq.at[pl.ds(0, B)], idst.at[0], sem_i[p])

    def copy_row(src2d, j, dstbuf):
        def body(i, _):
            dstbuf[pl.ds(i * L, L)] = src2d[j & 7, pl.ds(i * L, L)]
            return 0
        lax.fori_loop(0, B // L, body, 0, unroll=B // L)

    def fetch(j, slot):
        copy_row(isrc, j, isb[slot])
        copy_row(idst, j, idb[slot])
        pltpu.async_copy(a1.at[isb[slot]], arows.at[slot], sem_g[slot])
        pltpu.async_copy(b1.at[idb[slot]], brows.at[slot], sem_g[slot])

    def wait_fetch(j, slot):
        _wait(a1.at[isb[slot]], arows.at[slot], sem_g[slot])
        _wait(b1.at[idb[slot]], brows.at[slot], sem_g[slot])

    def compute(j, slot):
        _relu_add_rows(arows, brows, slot, B, H // L)

    def store(j, slot):
        pltpu.async_copy(arows.at[slot], e2_out.at[pl.ds(ebase + j * B, B)],
                         sem_s[slot])

    def wait_store(j, slot):
        _wait(arows.at[slot], e2_out.at[pl.ds(ebase + j * B, B)], sem_s[slot])

    def step(j, u, prev):
        @pl.when(j + D < nblk)
        def _():
            fetch_idx(j + D, u)

        @pl.when(j >= 1)
        def _():
            wait_store(j - 1, prev)

        @pl.when(j + D - 1 < nblk)
        def _():
            wait_idx(prev)
            fetch(j + D - 1, prev)
        wait_fetch(j, u)
        compute(j, u)
        store(j, u)

    for r in range(D):
        fetch_idx(r, r % 4)
    for b in range(D - 1):
        wait_idx(b % 4)
        fetch(b, b)

    def group(g, _):
        for t in range(D):
            j = D * g + t
            step(j, t, (t - 1) % D)
        return 0

    ngrp = nblk // D
    lax.fori_loop(0, ngrp, group, 0)
    for t in range(nblk - ngrp * D):
        j = ngrp * D + t
        step(jnp.int32(j), t, (t - 1) % D)
    wait_store(jnp.int32(nblk - 1), (nblk - 1) % D)


def _k3(E, H, a1, b1, srcq, dstq):
    mesh = plsc.VectorSubcoreMesh(core_axis_name="c", subcore_axis_name="s",
                                  num_cores=NC, num_subcores=NS)
    D = _D3
    kfn = pl.kernel(
        functools.partial(_k3_body, E, H),
        out_type=jax.ShapeDtypeStruct((E, H), jnp.float32),
        mesh=mesh,
        scratch_types=[
            pltpu.VMEM((8, B), jnp.int32),
            pltpu.VMEM((8, B), jnp.int32),
            pltpu.VMEM((D, B, H), jnp.float32),
            pltpu.VMEM((D, B, H), jnp.float32),
        ] + [pltpu.VMEM((B,), jnp.int32) for _ in range(2 * D)]
          + [pltpu.SemaphoreType.DMA for _ in range(4 + 2 * D)],
        name="egs_k3_edge_out",
    )
    return kfn(a1, b1, srcq, dstq)


def _tcA_kernel(DE, nf_ref, sn0_ref, sed_ref, wa_ref, ba_ref, we_ref, be_ref,
                h1_ref, a0_ref, b0_ref):
    D = nf_ref.shape[1]
    sed = sed_ref[0] + sed_ref[1]
    deg = sed[:, DE:DE + 1]
    inv = 1.0 / jnp.maximum(deg, 1.0)
    f32 = jnp.float32
    z = jnp.dot(nf_ref[...], wa_ref[0:D], preferred_element_type=f32)
    sn0 = sn0_ref[0] + sn0_ref[1]
    z += jnp.dot(sn0 * inv, wa_ref[D:2 * D], preferred_element_type=f32)
    z += jnp.dot(sed[:, 0:DE] * inv, wa_ref[2 * D:], preferred_element_type=f32)
    h1 = jnp.maximum(z + ba_ref[...], 0.0)
    h1_ref[...] = h1
    H = we_ref.shape[1]
    a0_ref[...] = jnp.dot(h1, we_ref[0:H], preferred_element_type=f32) + be_ref[...]
    b0_ref[...] = jnp.dot(h1, we_ref[H:], preferred_element_type=f32)


def _tcA(NP, DE, nf_p, sn0, sed, Wa0, ba0, We0, be0):
    H = We0.shape[1]
    RB = NP // 8
    row = pl.BlockSpec((RB, H), lambda i: (i, 0))
    part = pl.BlockSpec((2, RB, H), lambda i: (0, i, 0))
    return pl.pallas_call(
        functools.partial(_tcA_kernel, DE),
        grid=(NP // RB,),
        in_specs=[row, part, part,
                  pl.BlockSpec(Wa0.shape, lambda i: (0, 0)),
                  pl.BlockSpec(ba0.shape, lambda i: (0,)),
                  pl.BlockSpec(We0.shape, lambda i: (0, 0)),
                  pl.BlockSpec(be0.shape, lambda i: (0,))],
        out_specs=(row, row, row),
        out_shape=(jax.ShapeDtypeStruct((NP, H), jnp.float32),
                   jax.ShapeDtypeStruct((NP, H), jnp.float32),
                   jax.ShapeDtypeStruct((NP, H), jnp.float32)),
        name="egs_tcA",
    )(nf_p, sn0, sed, Wa0, ba0, We0, be0)


def _tcB_kernel(DE, h1_ref, sn1_ref, se1_ref, sed_ref, wa_ref, ba_ref, we_ref,
                be_ref, h2_ref, a1_ref, b1_ref):
    deg = (sed_ref[0] + sed_ref[1])[:, DE:DE + 1]
    inv = 1.0 / jnp.maximum(deg, 1.0)
    f32 = jnp.float32
    H = wa_ref.shape[1]
    z = jnp.dot(h1_ref[...], wa_ref[0:H], preferred_element_type=f32)
    sn1 = sn1_ref[0] + sn1_ref[1]
    z += jnp.dot(sn1 * inv, wa_ref[H:2 * H], preferred_element_type=f32)
    se1 = se1_ref[0] + se1_ref[1]
    z += jnp.dot(se1 * inv, wa_ref[2 * H:], preferred_element_type=f32)
    h2 = jnp.maximum(z + ba_ref[...], 0.0)
    h2_ref[...] = h2
    a1_ref[...] = jnp.dot(h2, we_ref[0:H], preferred_element_type=f32) + be_ref[...]
    b1_ref[...] = jnp.dot(h2, we_ref[H:], preferred_element_type=f32)


def _tcB(NP, DE, h1, sn1, se1, sed, Wa1, ba1, We1, be1):
    H = We1.shape[1]
    RB = NP // 8
    row = pl.BlockSpec((RB, H), lambda i: (i, 0))
    part = pl.BlockSpec((2, RB, H), lambda i: (0, i, 0))
    return pl.pallas_call(
        functools.partial(_tcB_kernel, DE),
        grid=(NP // RB,),
        in_specs=[row, part, part, part,
                  pl.BlockSpec(Wa1.shape, lambda i: (0, 0)),
                  pl.BlockSpec(ba1.shape, lambda i: (0,)),
                  pl.BlockSpec(We1.shape, lambda i: (0, 0)),
                  pl.BlockSpec(be1.shape, lambda i: (0,))],
        out_specs=(row, row, row),
        out_shape=(jax.ShapeDtypeStruct((NP, H), jnp.float32),
                   jax.ShapeDtypeStruct((NP, H), jnp.float32),
                   jax.ShapeDtypeStruct((NP, H), jnp.float32)),
        name="egs_tcB",
    )(h1, sn1, se1, sed, Wa1, ba1, We1, be1)


def kernel(nfeats, edge_index, efeats, Wa0, ba0, We0, be0, Wa1, ba1, We1, be1):
    N, D = nfeats.shape
    E = edge_index.shape[1]
    DE = efeats.shape[1]
    H = We0.shape[1]
    assert D == 128 and H == 128 and DE == 16
    NP = ((N + NS * 8 - 1) // (NS * 8)) * (NS * 8)
    assert E % (NC * NS * B) == 0

    srcq = edge_index[0]
    dstq = edge_index[1]
    nf_p = jnp.pad(nfeats, ((0, NP - N), (0, 0)))
    zn = jnp.zeros((NP, 128), jnp.float32)

    sn0, sed = _k1(NP, E, DE, nf_p, srcq, dstq, efeats, zn)
    h1, a0, b0 = _tcA(NP, DE, nf_p, sn0, sed, Wa0, ba0, We0, be0)
    sn1, se1 = _k2(NP, E, h1, a0, b0, srcq, dstq, zn)
    h2, a1, b1 = _tcB(NP, DE, h1, sn1, se1, sed, Wa1, ba1, We1, be1)
    e2 = _k3(E, H, a1, b1, srcq, dstq)
    return (h2[:N], e2)


# speedup vs baseline: 5.8825x; 1.0205x over previous
import functools

import jax
import jax.numpy as jnp
from jax import lax
from jax.experimental import pallas as pl
from jax.experimental.pallas import tpu as pltpu
from jax.experimental.pallas import tpu_sc as plsc

NC = 2
NS = 16
L = 16

B = 80


def _relu_add_rows(acc, other, slot, n_rows, n_col_vregs):
    def body(r, _):
        for q in range(n_col_vregs):
            a = acc[slot, r, pl.ds(q * L, L)]
            b = other[slot, r, pl.ds(q * L, L)]
            acc[slot, r, pl.ds(q * L, L)] = jnp.maximum(a + b, 0.0)
        return 0
    lax.fori_loop(0, n_rows, body, 0, unroll=2)


def _zero_acc(zn, acc, s, rpt):
    pltpu.sync_copy(zn.at[pl.ds(s * rpt, rpt)], acc.at[pl.ds(s * rpt, rpt)])


def _wait(src_dummy, dst_dummy, sem):
    pltpu.make_async_copy(src_dummy, dst_dummy, sem).wait()


def _pipe(nblk, D, fetch_idx, wait_idx, fetch, compute, scatter,
          wait_fetch, wait_scatter):
    def step(j, u, prev):
        @pl.when(j + D < nblk)
        def _():
            fetch_idx(j + D, u)

        @pl.when(j >= 1)
        def _():
            wait_scatter(j - 1, prev)

        @pl.when(j + D - 1 < nblk)
        def _():
            wait_idx(prev)
            fetch(j + D - 1, prev)
        wait_fetch(j, u)
        if compute is not None:
            compute(j, u)
        scatter(j, u)

    for r in range(D):
        fetch_idx(r, r)
    for b in range(D - 1):
        wait_idx(b)
        fetch(b, b)

    def group(g, _):
        for t in range(D):
            step(D * g + t, t, (t - 1) % D)
        return 0

    ngrp = nblk // D
    lax.fori_loop(0, ngrp, group, 0)
    for t in range(nblk - ngrp * D):
        step(jnp.int32(ngrp * D + t), t, (t - 1) % D)
    wait_scatter(jnp.int32(nblk - 1), (nblk - 1) % D)


def _k1_body(NP, E, DE, nf, srcq, dstq, ef, zn,
             sn0_out, sed_out,
             acc, isrc, idst, rows, erows, *rest):
    c = lax.axis_index("c")
    s = lax.axis_index("s")
    rpt = NP // NS
    epc = E // (NC * NS)
    nblk = epc // B
    ebase = c * (E // NC) + s * epc
    isb = rest[0:4]
    idb = rest[4:8]
    sem_i = rest[8:12]
    sem_g = rest[12:16]
    sem_s = rest[16:20]

    def fetch_idx(j, p):
        pltpu.async_copy(srcq.at[pl.ds(ebase + j * B, B)], isrc.at[j & 7], sem_i[p])
        pltpu.async_copy(dstq.at[pl.ds(ebase + j * B, B)], idst.at[j & 7], sem_i[p])

    def wait_idx(p):
        _wait(srcq.at[pl.ds(0, B)], isrc.at[0], sem_i[p])
        _wait(srcq.at[pl.ds(0, B)], idst.at[0], sem_i[p])

    def copy_row(src2d, j, dstbuf):
        def body(i, _):
            dstbuf[pl.ds(i * L, L)] = src2d[j & 7, pl.ds(i * L, L)]
            return 0
        lax.fori_loop(0, B // L, body, 0, unroll=B // L)

    def scatter(j, slot):
        pltpu.async_copy(rows.at[slot], acc.at[idb[slot]], sem_s[slot],
                         add=True)

    def wait_scatter(j, slot):
        _wait(rows.at[slot], acc.at[idb[slot]], sem_s[slot])

    _zero_acc(zn, acc, s, rpt)
    plsc.subcore_barrier()

    def fetch_a(j, slot):
        copy_row(isrc, j, isb[slot])
        copy_row(idst, j, idb[slot])
        pltpu.async_copy(nf.at[isb[slot]], rows.at[slot], sem_g[slot])

    def wait_fetch_a(j, slot):
        _wait(nf.at[isb[slot]], rows.at[slot], sem_g[slot])

    _pipe(nblk, 2, fetch_idx, wait_idx, fetch_a, None, scatter,
          wait_fetch_a, wait_scatter)

    plsc.subcore_barrier()
    pltpu.sync_copy(acc.at[pl.ds(s * rpt, rpt)],
                    sn0_out.at[c, pl.ds(s * rpt, rpt)])
    plsc.subcore_barrier()

    _zero_acc(zn, acc, s, rpt)
    one0 = jnp.where(lax.iota(jnp.int32, L) == 0, 1.0, 0.0).astype(jnp.float32)
    zv = jnp.zeros((L,), jnp.float32)

    def fill(r, _):
        for slot in (0, 1):
            rows[slot, r, pl.ds(DE, L)] = one0
            for q in range(DE // L + 1, 128 // L):
                rows[slot, r, pl.ds(q * L, L)] = zv
        return 0

    lax.fori_loop(0, B, fill, 0, unroll=2)
    plsc.subcore_barrier()

    def fetch_b(j, slot):
        copy_row(idst, j, idb[slot])
        pltpu.async_copy(ef.at[pl.ds(ebase + j * B, B)], erows.at[slot],
                         sem_g[slot])

    def wait_fetch_b(j, slot):
        _wait(ef.at[pl.ds(0, B)], erows.at[slot], sem_g[slot])

    def compute_b(j, slot):
        def cp(r, _):
            rows[slot, r, pl.ds(0, L)] = erows[slot, r, :]
            return 0
        lax.fori_loop(0, B, cp, 0, unroll=4)

    _pipe(nblk, 2, fetch_idx, wait_idx, fetch_b, compute_b, scatter,
          wait_fetch_b, wait_scatter)

    plsc.subcore_barrier()
    pltpu.sync_copy(acc.at[pl.ds(s * rpt, rpt)],
                    sed_out.at[c, pl.ds(s * rpt, rpt)])


def _k1(NP, E, DE, nf, srcq, dstq, ef, zn):
    mesh = plsc.VectorSubcoreMesh(core_axis_name="c", subcore_axis_name="s",
                                  num_cores=NC, num_subcores=NS)
    nblk = E // (NC * NS * B)
    kfn = pl.kernel(
        functools.partial(_k1_body, NP, E, DE),
        out_type=(jax.ShapeDtypeStruct((NC, NP, 128), jnp.float32),
                  jax.ShapeDtypeStruct((NC, NP, 128), jnp.float32)),
        mesh=mesh,
        scratch_types=[
            pltpu.VMEM_SHARED((NP, 128), jnp.float32),
            pltpu.VMEM((8, B), jnp.int32),
            pltpu.VMEM((8, B), jnp.int32),
            pltpu.VMEM((2, B, 128), jnp.float32),
            pltpu.VMEM((2, B, 16), jnp.float32),
        ] + [pltpu.VMEM((B,), jnp.int32) for _ in range(8)]
          + [pltpu.SemaphoreType.DMA for _ in range(12)],
        name="egs_k1_layer0_agg",
    )
    return kfn(nf, srcq, dstq, ef, zn)


def _k2_body(NP, E, h1t, a0t, b0t, srcq, dstq, zn,
             sn1_out, se1_out,
             acc, isrc, idst, arows, brows, *rest):
    c = lax.axis_index("c")
    s = lax.axis_index("s")
    rpt = NP // NS
    epc = E // (NC * NS)
    nblk = epc // B
    ebase = c * (E // NC) + s * epc
    isb = rest[0:4]
    idb = rest[4:8]
    sem_i = rest[8:12]
    sem_g = rest[12:16]
    sem_s = rest[16:20]

    def fetch_idx(j, p):
        pltpu.async_copy(srcq.at[pl.ds(ebase + j * B, B)], isrc.at[j & 7], sem_i[p])
        pltpu.async_copy(dstq.at[pl.ds(ebase + j * B, B)], idst.at[j & 7], sem_i[p])

    def wait_idx(p):
        _wait(srcq.at[pl.ds(0, B)], isrc.at[0], sem_i[p])
        _wait(srcq.at[pl.ds(0, B)], idst.at[0], sem_i[p])

    def copy_row(src2d, j, dstbuf):
        def body(i, _):
            dstbuf[pl.ds(i * L, L)] = src2d[j & 7, pl.ds(i * L, L)]
            return 0
        lax.fori_loop(0, B // L, body, 0, unroll=B // L)

    _zero_acc(zn, acc, s, rpt)
    plsc.subcore_barrier()
    abufs = (arows.at[0], arows.at[1], brows.at[0], brows.at[1])

    def fetch_a(j, slot):
        copy_row(isrc, j, isb[slot])
        copy_row(idst, j, idb[slot])
        pltpu.async_copy(h1t.at[isb[slot]], abufs[slot], sem_g[slot])

    def wait_fetch_a(j, slot):
        _wait(h1t.at[isb[slot]], abufs[slot], sem_g[slot])

    def scatter_a(j, slot):
        pltpu.async_copy(abufs[slot], acc.at[idb[slot]], sem_s[slot],
                         add=True)

    def wait_scatter_a(j, slot):
        _wait(abufs[slot], acc.at[idb[slot]], sem_s[slot])

    _pipe(nblk, 4, fetch_idx, wait_idx, fetch_a, None, scatter_a,
          wait_fetch_a, wait_scatter_a)

    plsc.subcore_barrier()
    pltpu.sync_copy(acc.at[pl.ds(s * rpt, rpt)],
                    sn1_out.at[c, pl.ds(s * rpt, rpt)])
    plsc.subcore_barrier()

    _zero_acc(zn, acc, s, rpt)
    plsc.subcore_barrier()

    def fetch_b(j, slot):
        copy_row(isrc, j, isb[slot])
        copy_row(idst, j, idb[slot])
        pltpu.async_copy(a0t.at[isb[slot]], arows.at[slot], sem_g[slot])
        pltpu.async_copy(b0t.at[idb[slot]], brows.at[slot], sem_g[slot])

    def wait_fetch_b(j, slot):
        _wait(a0t.at[isb[slot]], arows.at[slot], sem_g[slot])
        _wait(b0t.at[idb[slot]], brows.at[slot], sem_g[slot])

    def compute_b(j, slot):
        _relu_add_rows(arows, brows, slot, B, 8)

    def scatter_b(j, slot):
        pltpu.async_copy(arows.at[slot], acc.at[idb[slot]], sem_s[slot],
                         add=True)

    def wait_scatter_b(j, slot):
        _wait(arows.at[slot], acc.at[idb[slot]], sem_s[slot])

    _pipe(nblk, 2, fetch_idx, wait_idx, fetch_b, compute_b, scatter_b,
          wait_fetch_b, wait_scatter_b)

    plsc.subcore_barrier()
    pltpu.sync_copy(acc.at[pl.ds(s * rpt, rpt)],
                    se1_out.at[c, pl.ds(s * rpt, rpt)])


def _k2(NP, E, h1t, a0t, b0t, srcq, dstq, zn):
    mesh = plsc.VectorSubcoreMesh(core_axis_name="c", subcore_axis_name="s",
                                  num_cores=NC, num_subcores=NS)
    nblk = E // (NC * NS * B)
    kfn = pl.kernel(
        functools.partial(_k2_body, NP, E),
        out_type=(jax.ShapeDtypeStruct((NC, NP, 128), jnp.float32),
                  jax.ShapeDtypeStruct((NC, NP, 128), jnp.float32)),
        mesh=mesh,
        scratch_types=[
            pltpu.VMEM_SHARED((NP, 128), jnp.float32),
            pltpu.VMEM((8, B), jnp.int32),
            pltpu.VMEM((8, B), jnp.int32),
            pltpu.VMEM((2, B, 128), jnp.float32),
            pltpu.VMEM((2, B, 128), jnp.float32),
        ] + [pltpu.VMEM((B,), jnp.int32) for _ in range(8)]
          + [pltpu.SemaphoreType.DMA for _ in range(12)],
        name="egs_k2_layer1_agg",
    )
    return kfn(h1t, a0t, b0t, srcq, dstq, zn)


_D3 = 4


def _k3_body(E, H, a1, b1, srcq, dstq, e2_out,
             isrc, idst, arows, brows, *rest):
    c = lax.axis_index("c")
    s = lax.axis_index("s")
    wid = s * NC + c
    epw = E // (NC * NS)
    nblk = epw // B
    ebase = wid * epw
    D = _D3
    isb = rest[0:D]
    idb = rest[D:2 * D]
    sems = rest[2 * D:]
    sem_i = sems[0:4]
    sem_g = sems[4:4 + D]
    sem_s = sems[4 + D:4 + 2 * D]

    def fetch_idx(j, p):
        pltpu.async_copy(srcq.at[pl.ds(ebase + j * B, B)], isrc.at[j & 7], sem_i[p])
        pltpu.async_copy(dstq.at[pl.ds(ebase + j * B, B)], idst.at[j & 7], sem_i[p])

    def wait_idx(p):
        _wait(srcq.at[pl.ds(0, B)], isrc.at[0], sem_i[p])
        _wait(srcq.at[pl.ds(0, B)], idst.at[0], sem_i[p])

    def copy_row(src2d, j, dstbuf):
        def body(i, _):
            dstbuf[pl.ds(i * L, L)] = src2d[j & 7, pl.ds(i * L, L)]
            return 0
        lax.fori_loop(0, B // L, body, 0, unroll=B // L)

    def fetch(j, slot):
        copy_row(isrc, j, isb[slot])
        copy_row(idst, j, idb[slot])
        pltpu.async_copy(a1.at[isb[slot]], arows.at[slot], sem_g[slot])
        pltpu.async_copy(b1.at[idb[slot]], brows.at[slot], sem_g[slot])

    def wait_fetch(j, slot):
        _wait(a1.at[isb[slot]], arows.at[slot], sem_g[slot])
        _wait(b1.at[idb[slot]], brows.at[slot], sem_g[slot])

    def compute(j, slot):
        _relu_add_rows(arows, brows, slot, B, H // L)

    def store(j, slot):
        pltpu.async_copy(arows.at[slot], e2_out.at[pl.ds(ebase + j * B, B)],
                         sem_s[slot])

    def wait_store(j, slot):
        _wait(arows.at[slot], e2_out.at[pl.ds(ebase + j * B, B)], sem_s[slot])

    def step(j, u, prev):
        @pl.when(j + D < nblk)
        def _():
            fetch_idx(j + D, u)

        @pl.when(j >= 1)
        def _():
            wait_store(j - 1, prev)

        @pl.when(j + D - 1 < nblk)
        def _():
            wait_idx(prev)
            fetch(j + D - 1, prev)
        wait_fetch(j, u)
        compute(j, u)
        store(j, u)

    for r in range(D):
        fetch_idx(r, r % 4)
    for b in range(D - 1):
        wait_idx(b % 4)
        fetch(b, b)

    def group(g, _):
        for t in range(D):
            j = D * g + t
            step(j, t, (t - 1) % D)
        return 0

    ngrp = nblk // D
    lax.fori_loop(0, ngrp, group, 0)
    for t in range(nblk - ngrp * D):
        j = ngrp * D + t
        step(jnp.int32(j), t, (t - 1) % D)
    wait_store(jnp.int32(nblk - 1), (nblk - 1) % D)


def _k3(E, H, a1, b1, srcq, dstq):
    mesh = plsc.VectorSubcoreMesh(core_axis_name="c", subcore_axis_name="s",
                                  num_cores=NC, num_subcores=NS)
    D = _D3
    kfn = pl.kernel(
        functools.partial(_k3_body, E, H),
        out_type=jax.ShapeDtypeStruct((E, H), jnp.float32),
        mesh=mesh,
        scratch_types=[
            pltpu.VMEM((8, B), jnp.int32),
            pltpu.VMEM((8, B), jnp.int32),
            pltpu.VMEM((D, B, H), jnp.float32),
            pltpu.VMEM((D, B, H), jnp.float32),
        ] + [pltpu.VMEM((B,), jnp.int32) for _ in range(2 * D)]
          + [pltpu.SemaphoreType.DMA for _ in range(4 + 2 * D)],
        name="egs_k3_edge_out",
    )
    return kfn(a1, b1, srcq, dstq)


def _tcA_kernel(DE, nf_ref, sn0_ref, sed_ref, wa_ref, ba_ref, we_ref, be_ref,
                h1_ref, a0_ref, b0_ref):
    D = nf_ref.shape[1]
    sed = sed_ref[0] + sed_ref[1]
    deg = sed[:, DE:DE + 1]
    inv = 1.0 / jnp.maximum(deg, 1.0)
    f32 = jnp.float32
    z = jnp.dot(nf_ref[...], wa_ref[0:D], preferred_element_type=f32)
    sn0 = sn0_ref[0] + sn0_ref[1]
    z += jnp.dot(sn0 * inv, wa_ref[D:2 * D], preferred_element_type=f32)
    z += jnp.dot(sed[:, 0:DE] * inv, wa_ref[2 * D:], preferred_element_type=f32)
    h1 = jnp.maximum(z + ba_ref[...], 0.0)
    h1_ref[...] = h1
    H = we_ref.shape[1]
    a0_ref[...] = jnp.dot(h1, we_ref[0:H], preferred_element_type=f32) + be_ref[...]
    b0_ref[...] = jnp.dot(h1, we_ref[H:], preferred_element_type=f32)


def _tcA(NP, DE, nf_p, sn0, sed, Wa0, ba0, We0, be0):
    H = We0.shape[1]
    RB = NP // 8
    row = pl.BlockSpec((RB, H), lambda i: (i, 0))
    part = pl.BlockSpec((2, RB, H), lambda i: (0, i, 0))
    return pl.pallas_call(
        functools.partial(_tcA_kernel, DE),
        grid=(NP // RB,),
        in_specs=[row, part, part,
                  pl.BlockSpec(Wa0.shape, lambda i: (0, 0)),
                  pl.BlockSpec(ba0.shape, lambda i: (0,)),
                  pl.BlockSpec(We0.shape, lambda i: (0, 0)),
                  pl.BlockSpec(be0.shape, lambda i: (0,))],
        out_specs=(row, row, row),
        out_shape=(jax.ShapeDtypeStruct((NP, H), jnp.float32),
                   jax.ShapeDtypeStruct((NP, H), jnp.float32),
                   jax.ShapeDtypeStruct((NP, H), jnp.float32)),
        name="egs_tcA",
    )(nf_p, sn0, sed, Wa0, ba0, We0, be0)


def _tcB_kernel(DE, h1_ref, sn1_ref, se1_ref, sed_ref, wa_ref, ba_ref, we_ref,
                be_ref, h2_ref, a1_ref, b1_ref):
    deg = (sed_ref[0] + sed_ref[1])[:, DE:DE + 1]
    inv = 1.0 / jnp.maximum(deg, 1.0)
    f32 = jnp.float32
    H = wa_ref.shape[1]
    z = jnp.dot(h1_ref[...], wa_ref[0:H], preferred_element_type=f32)
    sn1 = sn1_ref[0] + sn1_ref[1]
    z += jnp.dot(sn1 * inv, wa_ref[H:2 * H], preferred_element_type=f32)
    se1 = se1_ref[0] + se1_ref[1]
    z += jnp.dot(se1 * inv, wa_ref[2 * H:], preferred_element_type=f32)
    h2 = jnp.maximum(z + ba_ref[...], 0.0)
    h2_ref[...] = h2
    a1_ref[...] = jnp.dot(h2, we_ref[0:H], preferred_element_type=f32) + be_ref[...]
    b1_ref[...] = jnp.dot(h2, we_ref[H:], preferred_element_type=f32)


def _tcB(NP, DE, h1, sn1, se1, sed, Wa1, ba1, We1, be1):
    H = We1.shape[1]
    RB = NP // 8
    row = pl.BlockSpec((RB, H), lambda i: (i, 0))
    part = pl.BlockSpec((2, RB, H), lambda i: (0, i, 0))
    return pl.pallas_call(
        functools.partial(_tcB_kernel, DE),
        grid=(NP // RB,),
        in_specs=[row, part, part, part,
                  pl.BlockSpec(Wa1.shape, lambda i: (0, 0)),
                  pl.BlockSpec(ba1.shape, lambda i: (0,)),
                  pl.BlockSpec(We1.shape, lambda i: (0, 0)),
                  pl.BlockSpec(be1.shape, lambda i: (0,))],
        out_specs=(row, row, row),
        out_shape=(jax.ShapeDtypeStruct((NP, H), jnp.float32),
                   jax.ShapeDtypeStruct((NP, H), jnp.float32),
                   jax.ShapeDtypeStruct((NP, H), jnp.float32)),
        name="egs_tcB",
    )(h1, sn1, se1, sed, Wa1, ba1, We1, be1)


def kernel(nfeats, edge_index, efeats, Wa0, ba0, We0, be0, Wa1, ba1, We1, be1):
    N, D = nfeats.shape
    E = edge_index.shape[1]
    DE = efeats.shape[1]
    H = We0.shape[1]
    assert D == 128 and H == 128 and DE == 16
    NP = ((N + NS * 8 - 1) // (NS * 8)) * (NS * 8)
    assert E % (NC * NS * B) == 0

    srcq = edge_index[0]
    dstq = edge_index[1]
    nf_p = jnp.pad(nfeats, ((0, NP - N), (0, 0)))
    zn = jnp.zeros((NP, 128), jnp.float32)

    sn0, sed = _k1(NP, E, DE, nf_p, srcq, dstq, efeats, zn)
    h1, a0, b0 = _tcA(NP, DE, nf_p, sn0, sed, Wa0, ba0, We0, be0)
    sn1, se1 = _k2(NP, E, h1, a0, b0, srcq, dstq, zn)
    h2, a1, b1 = _tcB(NP, DE, h1, sn1, se1, sed, Wa1, ba1, We1, be1)
    e2 = _k3(E, H, a1, b1, srcq, dstq)
    return (h2[:N], e2)
